# baseline (speedup 1.0000x reference)
import jax
import jax.numpy as jnp
from jax.experimental import pallas as pl
from jax.experimental.pallas import tpu as pltpu


_NSEM = 16


def _gather_body(idx_ref, in_hbm, out_hbm, stage, in_sems, out_sem):
    nb = out_hbm.shape[0]

    def start(b, carry):
        pltpu.make_async_copy(
            in_hbm.at[b, idx_ref[b]], stage.at[b], in_sems.at[b % _NSEM]
        ).start()
        return carry

    jax.lax.fori_loop(0, nb, start, 0)

    def drain(b, carry):
        pltpu.make_async_copy(
            in_hbm.at[0, 0], stage.at[0], in_sems.at[b % _NSEM]
        ).wait()
        return carry

    jax.lax.fori_loop(0, nb, drain, 0)

    out_copy = pltpu.make_async_copy(stage, out_hbm, out_sem)
    out_copy.start()
    out_copy.wait()


def kernel(inputs, idx_x):
    b, n, h, w = inputs.shape
    gather_idx = (n - idx_x).astype(jnp.int32)

    grid_spec = pltpu.PrefetchScalarGridSpec(
        num_scalar_prefetch=1,
        grid=(1,),
        in_specs=[pl.BlockSpec(memory_space=pltpu.MemorySpace.HBM)],
        out_specs=pl.BlockSpec(memory_space=pltpu.MemorySpace.HBM),
        scratch_shapes=[
            pltpu.VMEM((b, h, w), jnp.float32),
            pltpu.SemaphoreType.DMA((_NSEM,)),
            pltpu.SemaphoreType.DMA,
        ],
    )
    return pl.pallas_call(
        _gather_body,
        grid_spec=grid_spec,
        out_shape=jax.ShapeDtypeStruct((b, h, w), inputs.dtype),
    )(gather_idx, inputs)


# speedup vs baseline: 1.0029x; 1.0029x over previous
import jax
import jax.numpy as jnp
from jax.experimental import pallas as pl
from jax.experimental.pallas import tpu as pltpu


_NSEM = 16


def _gather_body(idx_ref, in_hbm, out_hbm, stage, in_sems, out_sem):
    nb = out_hbm.shape[0]

    def start(b, carry):
        pltpu.make_async_copy(
            in_hbm.at[b, idx_ref[b]], stage.at[b], in_sems.at[0]
        ).start()
        return carry

    jax.lax.fori_loop(0, nb, start, 0)

    pltpu.make_async_copy(
        in_hbm.at[pl.ds(0, nb), 0], stage, in_sems.at[0]
    ).wait()

    out_copy = pltpu.make_async_copy(stage, out_hbm, out_sem)
    out_copy.start()
    out_copy.wait()


def kernel(inputs, idx_x):
    b, n, h, w = inputs.shape
    gather_idx = (n - idx_x).astype(jnp.int32)

    grid_spec = pltpu.PrefetchScalarGridSpec(
        num_scalar_prefetch=1,
        grid=(1,),
        in_specs=[pl.BlockSpec(memory_space=pltpu.MemorySpace.HBM)],
        out_specs=pl.BlockSpec(memory_space=pltpu.MemorySpace.HBM),
        scratch_shapes=[
            pltpu.VMEM((b, h, w), jnp.float32),
            pltpu.SemaphoreType.DMA((_NSEM,)),
            pltpu.SemaphoreType.DMA,
        ],
    )
    return pl.pallas_call(
        _gather_body,
        grid_spec=grid_spec,
        out_shape=jax.ShapeDtypeStruct((b, h, w), inputs.dtype),
    )(gather_idx, inputs)
